# baseline (speedup 1.0000x reference)
import functools

import jax
import jax.numpy as jnp
from jax import lax
from jax.experimental import pallas as pl
from jax.experimental.pallas import tpu as pltpu
from jax.experimental.pallas import tpu_sc as plsc

P, L, E, D = 50000, 10000, 160000, 32
N_ITERS = 4

NC, NS = 2, 16
NW = NC * NS
EP = 163840
EHA = 81920
EHB = EP - EHA
GCHUNK = 512
CHUNK = 1024
EPT = EP // NS

PH = 25008
PACCH = PH + 16
PHT = PACCH // NS
PRT = PH // NS
LH = 5008
LACCH = LH + 16
LHT = LACCH // NS
LRT = LH // NS


def _make_gather_body(start_edge, nblk):
    def body(lst, pst, li2d, pi2d, ls_out, ps_out, idxb, b0, b1, b2,
             lt_sh, isem, g0, g1, g2, s0, s1, s2):
        c = lax.axis_index("c")
        s = lax.axis_index("s")
        w = c * NS + s
        base = w * (nblk * GCHUNK)
        dbuf = (b0, b1, b2)
        gsem = (g0, g1, g2)
        ssem = (s0, s1, s2)
        st = pltpu.async_copy(lst.at[pl.ds(s * (L // NS), L // NS)],
                              lt_sh.at[pl.ds(s * (L // NS), L // NS)], isem)
        irow = start_edge // GCHUNK + w * nblk
        d1 = pltpu.async_copy(li2d.at[pl.ds(irow, nblk)],
                              idxb.at[pl.ds(0, nblk)], isem)
        d2 = pltpu.async_copy(pi2d.at[pl.ds(irow, nblk)],
                              idxb.at[pl.ds(nblk, nblk)], isem)
        st.wait()
        d1.wait()
        d2.wait()
        plsc.subcore_barrier()
        n = 2 * nblk
        plan = [(lt_sh, ls_out, j) for j in range(nblk)] + \
               [(pst, ps_out, nblk + j) for j in range(nblk)]
        gd, sd = [], []
        for k, (tab, out, row) in enumerate(plan):
            sl = k % 3
            if k >= 3:
                sd[k - 3].wait()
            gd.append(pltpu.async_copy(tab.at[idxb.at[row]], dbuf[sl],
                                       gsem[sl]))
            if k >= 1:
                gd[k - 1].wait()
                off = base + ((k - 1) % nblk) * GCHUNK
                sd.append(pltpu.async_copy(dbuf[(k - 1) % 3],
                                           plan[k - 1][1].at[pl.ds(off,
                                                                   GCHUNK)],
                                           ssem[(k - 1) % 3]))
        gd[-1].wait()
        off = base + (nblk - 1) * GCHUNK
        sd.append(pltpu.async_copy(dbuf[(n - 1) % 3],
                                   ps_out.at[pl.ds(off, GCHUNK)],
                                   ssem[(n - 1) % 3]))
        for k in (n - 3, n - 2, n - 1):
            sd[k].wait()
    return body


def _make_scatter_body(acc_rows, own, tile_zero, tile_read):
    del acc_rows

    def body(mX, idx2, zz, out, i0, i1, m0, m1b, acc,
             is0, is1, l0, l1, c0, c1):
        c = lax.axis_index("c")
        s = lax.axis_index("s")
        ibuf = (i0, i1)
        mbuf = (m0, m1b)
        isem = (is0, is1)
        lsem = (l0, l1)
        csem = (c0, c1)
        pltpu.sync_copy(zz, m0)
        z = 0
        while z < tile_zero:
            step = min(CHUNK, tile_zero - z)
            pltpu.sync_copy(m0.at[pl.ds(0, step)],
                            acc.at[pl.ds(s * tile_zero + z, step)])
            z += step
        plsc.subcore_barrier()
        n = EHA // NS // CHUNK
        moffs = [s * (EHA // NS) + j * CHUNK for j in range(n)]
        srcs = [mX] * n
        ioffs = [c * EHA + moffs[j] for j in range(n)]
        ld = [pltpu.async_copy(srcs[0].at[pl.ds(moffs[0], CHUNK)], m0, l0)]
        ix = [pltpu.async_copy(idx2.at[pl.ds(ioffs[0], CHUNK)], i0, is0)]
        sc = []
        for k in range(n):
            sl = k % 2
            ld[k].wait()
            ix[k].wait()
            sc.append(pltpu.async_copy(
                mbuf[sl], acc.at[plsc.Indices(ibuf[sl], ignored_value=own)],
                csem[sl], add=True))
            if k >= 1:
                sc[k - 1].wait()
            if k + 1 < n:
                nsl = (k + 1) % 2
                ld.append(pltpu.async_copy(
                    srcs[k + 1].at[pl.ds(moffs[k + 1], CHUNK)], mbuf[nsl],
                    lsem[nsl]))
                ix.append(pltpu.async_copy(
                    idx2.at[pl.ds(ioffs[k + 1], CHUNK)], ibuf[nsl],
                    isem[nsl]))
        sc[-1].wait()
        plsc.subcore_barrier()
        z = 0
        while z < tile_read:
            step = min(CHUNK, tile_read - z)
            pltpu.sync_copy(acc.at[pl.ds(s * tile_read + z, step)],
                            m1b.at[pl.ds(0, step)])
            pltpu.sync_copy(m1b.at[pl.ds(0, step)],
                            out.at[pl.ds(c * own + s * tile_read + z, step)])
            z += step
    return body


@functools.cache
def _sc_kernels():
    mesh = plsc.VectorSubcoreMesh(core_axis_name="c", subcore_axis_name="s",
                                  num_cores=NC, num_subcores=NS)
    params = pltpu.CompilerParams(use_tc_tiling_on_sc=False)
    def gather_scratch(nblk):
        return (
            pltpu.VMEM((2 * nblk, GCHUNK), jnp.int32),
            pltpu.VMEM((GCHUNK, D), jnp.float32),
            pltpu.VMEM((GCHUNK, D), jnp.float32),
            pltpu.VMEM((GCHUNK, D), jnp.float32),
            pltpu.VMEM_SHARED((L, D), jnp.float32),
        ) + (pltpu.SemaphoreType.DMA,) * 7

    gathers = tuple(
        pl.kernel(
            _make_gather_body(start, rows // (NW * GCHUNK)),
            out_type=(jax.ShapeDtypeStruct((rows, D), jnp.float32),
                      jax.ShapeDtypeStruct((rows, D), jnp.float32)),
            mesh=mesh,
            scratch_types=gather_scratch(rows // (NW * GCHUNK)),
            compiler_params=params,
        ) for start, rows in ((0, EHA), (EHA, EHB)))

    def scatter_kernel(acc_rows, own, tile_zero, tile_read, out_rows):
        return pl.kernel(
            _make_scatter_body(acc_rows, own, tile_zero, tile_read),
            out_type=jax.ShapeDtypeStruct((out_rows, D), jnp.float32),
            mesh=mesh,
            scratch_types=(
                pltpu.VMEM((CHUNK,), jnp.int32),
                pltpu.VMEM((CHUNK,), jnp.int32),
                pltpu.VMEM((CHUNK, D), jnp.float32),
                pltpu.VMEM((CHUNK, D), jnp.float32),
                pltpu.VMEM_SHARED((acc_rows, D), jnp.float32),
            ) + (pltpu.SemaphoreType.DMA,) * 6,
            compiler_params=params,
        )

    scatter_p = scatter_kernel(PACCH, PH, PHT, PRT, NC * PH)
    scatter_l = scatter_kernel(LACCH, LH, LHT, LRT, NC * LH)
    return gathers[0], gathers[1], scatter_p, scatter_l


_EBLK = 8192


def _mlp_body(ls, ps, w1, b1, w2, b2, w3, b3, m1, m2):
    x1 = jnp.concatenate([ls[...], ps[...]], axis=1)
    x2 = jnp.concatenate([ps[...], ls[...]], axis=1)
    h = jnp.concatenate([x1, x2], axis=0)
    h = jnp.maximum(jnp.dot(h, w1[...], preferred_element_type=jnp.float32)
                    + b1[...], 0.0)
    h = jnp.maximum(jnp.dot(h, w2[...], preferred_element_type=jnp.float32)
                    + b2[...], 0.0)
    m = jnp.dot(h, w3[...], preferred_element_type=jnp.float32) + b3[...]
    m1[...] = m[:_EBLK]
    m2[...] = m[_EBLK:]


def _mlp(ls, ps, w1, b1, w2, b2, w3, b3):
    rows = ls.shape[0]
    full = lambda shape: pl.BlockSpec(shape, lambda i: (0,) * len(shape))
    eb = pl.BlockSpec((_EBLK, D), lambda i: (i, 0))
    return pl.pallas_call(
        _mlp_body,
        grid=(rows // _EBLK,),
        in_specs=[eb, eb, full((2 * D, 256)), full((1, 256)),
                  full((256, 256)), full((1, 256)), full((256, D)),
                  full((1, D))],
        out_specs=[eb, eb],
        out_shape=(jax.ShapeDtypeStruct((rows, D), jnp.float32),
                   jax.ShapeDtypeStruct((rows, D), jnp.float32)),
    )(ls, ps, w1, b1.reshape(1, -1), w2, b2.reshape(1, -1), w3,
      b3.reshape(1, -1))


def _gru_body(aggA, aggB, h, wi_r, wi_z, wi_n, wh_r, wh_z, wh_n,
              bi_r, bi_z, bi_n, bh_r, bh_z, bh_n, out):
    x = aggA[...] + aggB[...]
    hh = h[...]
    dot = lambda a, b: jnp.dot(a, b[...], preferred_element_type=jnp.float32)
    r = jax.nn.sigmoid(dot(x, wi_r) + bi_r[...] + dot(hh, wh_r) + bh_r[...])
    z = jax.nn.sigmoid(dot(x, wi_z) + bi_z[...] + dot(hh, wh_z) + bh_z[...])
    n = jnp.tanh(dot(x, wi_n) + bi_n[...] + r * (dot(hh, wh_n) + bh_n[...]))
    out[...] = (1.0 - z) * n + z * hh


def _gru(aggA, aggB, h, wih, whh, bih, bhh, nrows, blk):
    full = lambda shape: pl.BlockSpec(shape, lambda i: (0,) * len(shape))
    wspec = [full((D, D))] * 6 + [full((1, D))] * 6
    ws = ([wih[:, :D], wih[:, D:2 * D], wih[:, 2 * D:],
           whh[:, :D], whh[:, D:2 * D], whh[:, 2 * D:]]
          + [b.reshape(1, -1) for b in
             (bih[:D], bih[D:2 * D], bih[2 * D:],
              bhh[:D], bhh[D:2 * D], bhh[2 * D:])])
    rb = pl.BlockSpec((blk, D), lambda i: (i, 0))
    return pl.pallas_call(
        _gru_body,
        grid=(nrows // blk,),
        in_specs=[rb, rb, rb] + wspec,
        out_specs=rb,
        out_shape=jax.ShapeDtypeStruct((nrows, D), jnp.float32),
    )(aggA, aggB, h, *ws)


def _readout_body(h, w1, b1, w2, b2, w3, b3, out):
    dot = lambda a, b: jnp.dot(a, b[...], preferred_element_type=jnp.float32)
    r = jnp.maximum(dot(h[...], w1) + b1[...], 0.0)
    r = jnp.maximum(dot(r, w2) + b2[...], 0.0)
    out[...] = dot(r, w3) + b3[...]


def _readout(h, w1, b1, w2, b2, w3, b3, blk=5000):
    full = lambda shape: pl.BlockSpec(shape, lambda i: (0,) * len(shape))
    return pl.pallas_call(
        _readout_body,
        grid=(P // blk,),
        in_specs=[pl.BlockSpec((blk, D), lambda i: (i, 0)),
                  full((D, 256)), full((1, 256)), full((256, 256)),
                  full((1, 256)), full((256, 1)), full((1, 1))],
        out_specs=pl.BlockSpec((blk, 1), lambda i: (i, 0)),
        out_shape=jax.ShapeDtypeStruct((P, 1), jnp.float32),
    )(h, w1, b1.reshape(1, -1), w2, b2.reshape(1, -1), w3, b3.reshape(1, -1))


def kernel(TM, link_capacity, link_indices, path_indices, sequ_indices,
           n_paths, n_links, n_total, paths,
           msg_W1, msg_b1, msg_W2, msg_b2, msg_W3, msg_b3,
           ro_W1, ro_b1, ro_W2, ro_b2, ro_W3, ro_b3,
           path_Wih, path_Whh, path_bih, path_bhh,
           link_Wih, link_Whh, link_bih, link_bhh):
    pad = EP - E
    li_g = jnp.concatenate([link_indices, jnp.zeros((pad,), jnp.int32)])
    pi_g = jnp.concatenate([path_indices, jnp.zeros((pad,), jnp.int32)])
    li_s = jnp.concatenate([link_indices, jnp.full((pad,), L, jnp.int32)])
    pi_s = jnp.concatenate([path_indices, jnp.full((pad,), P, jnp.int32)])
    pi_r = (jnp.where(pi_s < PH, pi_s, PH),
            jnp.where((pi_s >= PH) & (pi_s < 2 * PH), pi_s - PH, PH))
    li_r = (jnp.where(li_s < LH, li_s, LH),
            jnp.where((li_s >= LH) & (li_s < 2 * LH), li_s - LH, LH))
    pi_sA = jnp.concatenate([x[:EHA] for x in pi_r])
    pi_sB = jnp.concatenate([x[EHA:] for x in pi_r])
    li_sA = jnp.concatenate([x[:EHA] for x in li_r])
    li_sB = jnp.concatenate([x[EHA:] for x in li_r])
    li_g = li_g.reshape(EP // GCHUNK, GCHUNK)
    pi_g = pi_g.reshape(EP // GCHUNK, GCHUNK)
    zz = jnp.zeros((CHUNK, D), jnp.float32)

    link_states = jnp.zeros((L, D), jnp.float32).at[:, 0].set(link_capacity)
    path_states = jnp.zeros((P, D), jnp.float32).at[:, 0].set(
        jnp.reshape(TM, (-1,)))

    gather_a, gather_b, scatter_p, scatter_l = _sc_kernels()
    for it in range(N_ITERS):
        lsA, psA = gather_a(link_states, path_states, li_g, pi_g)
        lsB, psB = gather_b(link_states, path_states, li_g, pi_g)
        m1A, m2A = _mlp(lsA, psA, msg_W1, msg_b1, msg_W2, msg_b2,
                        msg_W3, msg_b3)
        m1B, m2B = _mlp(lsB, psB, msg_W1, msg_b1, msg_W2, msg_b2,
                        msg_W3, msg_b3)
        paA = scatter_p(m1A, pi_sA, zz)
        paB = scatter_p(m1B, pi_sB, zz)
        path_states = _gru(paA, paB, path_states, path_Wih, path_Whh,
                           path_bih, path_bhh, P, 10000)
        if it < N_ITERS - 1:
            laA = scatter_l(m2A, li_sA, zz)
            laB = scatter_l(m2B, li_sB, zz)
            link_states = _gru(laA, laB, link_states, link_Wih, link_Whh,
                               link_bih, link_bhh, L, 10000)
    return _readout(path_states, ro_W1, ro_b1, ro_W2, ro_b2, ro_W3, ro_b3,
                    blk=10000)


# speedup vs baseline: 1.0425x; 1.0425x over previous
import functools

import jax
import jax.numpy as jnp
from jax import lax
from jax.experimental import pallas as pl
from jax.experimental.pallas import tpu as pltpu
from jax.experimental.pallas import tpu_sc as plsc

P, L, E, D = 50000, 10000, 160000, 32
N_ITERS = 4

NC, NS = 2, 16
NW = NC * NS
EP = 163840
EHA = 81920
EHB = EP - EHA
GCHUNK = 512
CHUNK = 1024
EPT = EP // NS

PH = 25008
PACCH = PH + 16
PHT = PACCH // NS
PRT = PH // NS
LH = 5008
LACCH = LH + 16
LHT = LACCH // NS
LRT = LH // NS


def _make_gather_body(start_edge, nblk):
    def body(lst, pst, li2d, pi2d, ls_out, ps_out, idxb, b0, b1, b2,
             lt_sh, isem, g0, g1, g2, s0, s1, s2):
        c = lax.axis_index("c")
        s = lax.axis_index("s")
        w = c * NS + s
        base = w * (nblk * GCHUNK)
        dbuf = (b0, b1, b2)
        gsem = (g0, g1, g2)
        ssem = (s0, s1, s2)
        st = pltpu.async_copy(lst.at[pl.ds(s * (L // NS), L // NS)],
                              lt_sh.at[pl.ds(s * (L // NS), L // NS)], isem)
        irow = start_edge // GCHUNK + w * nblk
        d1 = pltpu.async_copy(li2d.at[pl.ds(irow, nblk)],
                              idxb.at[pl.ds(0, nblk)], isem)
        d2 = pltpu.async_copy(pi2d.at[pl.ds(irow, nblk)],
                              idxb.at[pl.ds(nblk, nblk)], isem)
        st.wait()
        d1.wait()
        d2.wait()
        plsc.subcore_barrier()
        n = 2 * nblk
        plan = [(lt_sh, ls_out, j) for j in range(nblk)] + \
               [(pst, ps_out, nblk + j) for j in range(nblk)]
        gd, sd = [], []
        for k, (tab, out, row) in enumerate(plan):
            sl = k % 3
            if k >= 3:
                sd[k - 3].wait()
            gd.append(pltpu.async_copy(tab.at[idxb.at[row]], dbuf[sl],
                                       gsem[sl]))
            if k >= 1:
                gd[k - 1].wait()
                off = base + ((k - 1) % nblk) * GCHUNK
                sd.append(pltpu.async_copy(dbuf[(k - 1) % 3],
                                           plan[k - 1][1].at[pl.ds(off,
                                                                   GCHUNK)],
                                           ssem[(k - 1) % 3]))
        gd[-1].wait()
        off = base + (nblk - 1) * GCHUNK
        sd.append(pltpu.async_copy(dbuf[(n - 1) % 3],
                                   ps_out.at[pl.ds(off, GCHUNK)],
                                   ssem[(n - 1) % 3]))
        for k in (n - 3, n - 2, n - 1):
            sd[k].wait()
    return body


def _make_scatter_body(acc_rows, own, tile_zero, tile_read):
    del acc_rows

    def body(mA, mB, idx2, zz, out, i0, i1, m0, m1b, acc,
             is0, is1, l0, l1, c0, c1):
        c = lax.axis_index("c")
        s = lax.axis_index("s")
        ibuf = (i0, i1)
        mbuf = (m0, m1b)
        isem = (is0, is1)
        lsem = (l0, l1)
        csem = (c0, c1)
        pltpu.sync_copy(zz, m0)
        z = 0
        while z < tile_zero:
            step = min(CHUNK, tile_zero - z)
            pltpu.sync_copy(m0.at[pl.ds(0, step)],
                            acc.at[pl.ds(s * tile_zero + z, step)])
            z += step
        plsc.subcore_barrier()
        n = EPT // CHUNK
        ka = EHA // NS // CHUNK
        moffs = [s * (EHA // NS) + j * CHUNK if j < ka
                 else s * (EHB // NS) + (j - ka) * CHUNK for j in range(n)]
        srcs = [mA if j < ka else mB for j in range(n)]
        ioffs = [c * EP + (0 if j < ka else EHA) + moffs[j]
                 for j in range(n)]
        ld = [pltpu.async_copy(srcs[0].at[pl.ds(moffs[0], CHUNK)], m0, l0)]
        ix = [pltpu.async_copy(idx2.at[pl.ds(ioffs[0], CHUNK)], i0, is0)]
        sc = []
        for k in range(n):
            sl = k % 2
            ld[k].wait()
            ix[k].wait()
            sc.append(pltpu.async_copy(
                mbuf[sl], acc.at[plsc.Indices(ibuf[sl], ignored_value=own)],
                csem[sl], add=True))
            if k >= 1:
                sc[k - 1].wait()
            if k + 1 < n:
                nsl = (k + 1) % 2
                ld.append(pltpu.async_copy(
                    srcs[k + 1].at[pl.ds(moffs[k + 1], CHUNK)], mbuf[nsl],
                    lsem[nsl]))
                ix.append(pltpu.async_copy(
                    idx2.at[pl.ds(ioffs[k + 1], CHUNK)], ibuf[nsl],
                    isem[nsl]))
        sc[-1].wait()
        plsc.subcore_barrier()
        z = 0
        while z < tile_read:
            step = min(CHUNK, tile_read - z)
            pltpu.sync_copy(acc.at[pl.ds(s * tile_read + z, step)],
                            m1b.at[pl.ds(0, step)])
            pltpu.sync_copy(m1b.at[pl.ds(0, step)],
                            out.at[pl.ds(c * own + s * tile_read + z, step)])
            z += step
    return body


@functools.cache
def _sc_kernels():
    mesh = plsc.VectorSubcoreMesh(core_axis_name="c", subcore_axis_name="s",
                                  num_cores=NC, num_subcores=NS)
    params = pltpu.CompilerParams(use_tc_tiling_on_sc=False)
    def gather_scratch(nblk):
        return (
            pltpu.VMEM((2 * nblk, GCHUNK), jnp.int32),
            pltpu.VMEM((GCHUNK, D), jnp.float32),
            pltpu.VMEM((GCHUNK, D), jnp.float32),
            pltpu.VMEM((GCHUNK, D), jnp.float32),
            pltpu.VMEM_SHARED((L, D), jnp.float32),
        ) + (pltpu.SemaphoreType.DMA,) * 7

    gathers = tuple(
        pl.kernel(
            _make_gather_body(start, rows // (NW * GCHUNK)),
            out_type=(jax.ShapeDtypeStruct((rows, D), jnp.float32),
                      jax.ShapeDtypeStruct((rows, D), jnp.float32)),
            mesh=mesh,
            scratch_types=gather_scratch(rows // (NW * GCHUNK)),
            compiler_params=params,
        ) for start, rows in ((0, EHA), (EHA, EHB)))

    def scatter_kernel(acc_rows, own, tile_zero, tile_read, out_rows):
        return pl.kernel(
            _make_scatter_body(acc_rows, own, tile_zero, tile_read),
            out_type=jax.ShapeDtypeStruct((out_rows, D), jnp.float32),
            mesh=mesh,
            scratch_types=(
                pltpu.VMEM((CHUNK,), jnp.int32),
                pltpu.VMEM((CHUNK,), jnp.int32),
                pltpu.VMEM((CHUNK, D), jnp.float32),
                pltpu.VMEM((CHUNK, D), jnp.float32),
                pltpu.VMEM_SHARED((acc_rows, D), jnp.float32),
            ) + (pltpu.SemaphoreType.DMA,) * 6,
            compiler_params=params,
        )

    scatter_p = scatter_kernel(PACCH, PH, PHT, PRT, NC * PH)
    scatter_l = scatter_kernel(LACCH, LH, LHT, LRT, NC * LH)
    return gathers[0], gathers[1], scatter_p, scatter_l


_EBLK = 8192


def _mlp_body(ls, ps, w1, b1, w2, b2, w3, b3, m1, m2):
    x1 = jnp.concatenate([ls[...], ps[...]], axis=1)
    x2 = jnp.concatenate([ps[...], ls[...]], axis=1)
    h = jnp.concatenate([x1, x2], axis=0)
    h = jnp.maximum(jnp.dot(h, w1[...], preferred_element_type=jnp.float32)
                    + b1[...], 0.0)
    h = jnp.maximum(jnp.dot(h, w2[...], preferred_element_type=jnp.float32)
                    + b2[...], 0.0)
    m = jnp.dot(h, w3[...], preferred_element_type=jnp.float32) + b3[...]
    m1[...] = m[:_EBLK]
    m2[...] = m[_EBLK:]


def _mlp(ls, ps, w1, b1, w2, b2, w3, b3):
    rows = ls.shape[0]
    full = lambda shape: pl.BlockSpec(shape, lambda i: (0,) * len(shape))
    eb = pl.BlockSpec((_EBLK, D), lambda i: (i, 0))
    return pl.pallas_call(
        _mlp_body,
        grid=(rows // _EBLK,),
        in_specs=[eb, eb, full((2 * D, 256)), full((1, 256)),
                  full((256, 256)), full((1, 256)), full((256, D)),
                  full((1, D))],
        out_specs=[eb, eb],
        out_shape=(jax.ShapeDtypeStruct((rows, D), jnp.float32),
                   jax.ShapeDtypeStruct((rows, D), jnp.float32)),
    )(ls, ps, w1, b1.reshape(1, -1), w2, b2.reshape(1, -1), w3,
      b3.reshape(1, -1))


def _gru_body(agg, h, wi_r, wi_z, wi_n, wh_r, wh_z, wh_n,
              bi_r, bi_z, bi_n, bh_r, bh_z, bh_n, out):
    x = agg[...]
    hh = h[...]
    dot = lambda a, b: jnp.dot(a, b[...], preferred_element_type=jnp.float32)
    r = jax.nn.sigmoid(dot(x, wi_r) + bi_r[...] + dot(hh, wh_r) + bh_r[...])
    z = jax.nn.sigmoid(dot(x, wi_z) + bi_z[...] + dot(hh, wh_z) + bh_z[...])
    n = jnp.tanh(dot(x, wi_n) + bi_n[...] + r * (dot(hh, wh_n) + bh_n[...]))
    out[...] = (1.0 - z) * n + z * hh


def _gru(agg, h, wih, whh, bih, bhh, nrows, blk):
    full = lambda shape: pl.BlockSpec(shape, lambda i: (0,) * len(shape))
    wspec = [full((D, D))] * 6 + [full((1, D))] * 6
    ws = ([wih[:, :D], wih[:, D:2 * D], wih[:, 2 * D:],
           whh[:, :D], whh[:, D:2 * D], whh[:, 2 * D:]]
          + [b.reshape(1, -1) for b in
             (bih[:D], bih[D:2 * D], bih[2 * D:],
              bhh[:D], bhh[D:2 * D], bhh[2 * D:])])
    rb = pl.BlockSpec((blk, D), lambda i: (i, 0))
    return pl.pallas_call(
        _gru_body,
        grid=(nrows // blk,),
        in_specs=[rb, rb] + wspec,
        out_specs=rb,
        out_shape=jax.ShapeDtypeStruct((nrows, D), jnp.float32),
    )(agg, h, *ws)


def _readout_body(h, w1, b1, w2, b2, w3, b3, out):
    dot = lambda a, b: jnp.dot(a, b[...], preferred_element_type=jnp.float32)
    r = jnp.maximum(dot(h[...], w1) + b1[...], 0.0)
    r = jnp.maximum(dot(r, w2) + b2[...], 0.0)
    out[...] = dot(r, w3) + b3[...]


def _readout(h, w1, b1, w2, b2, w3, b3, blk=5000):
    full = lambda shape: pl.BlockSpec(shape, lambda i: (0,) * len(shape))
    return pl.pallas_call(
        _readout_body,
        grid=(P // blk,),
        in_specs=[pl.BlockSpec((blk, D), lambda i: (i, 0)),
                  full((D, 256)), full((1, 256)), full((256, 256)),
                  full((1, 256)), full((256, 1)), full((1, 1))],
        out_specs=pl.BlockSpec((blk, 1), lambda i: (i, 0)),
        out_shape=jax.ShapeDtypeStruct((P, 1), jnp.float32),
    )(h, w1, b1.reshape(1, -1), w2, b2.reshape(1, -1), w3, b3.reshape(1, -1))


def kernel(TM, link_capacity, link_indices, path_indices, sequ_indices,
           n_paths, n_links, n_total, paths,
           msg_W1, msg_b1, msg_W2, msg_b2, msg_W3, msg_b3,
           ro_W1, ro_b1, ro_W2, ro_b2, ro_W3, ro_b3,
           path_Wih, path_Whh, path_bih, path_bhh,
           link_Wih, link_Whh, link_bih, link_bhh):
    pad = EP - E
    li_g = jnp.concatenate([link_indices, jnp.zeros((pad,), jnp.int32)])
    pi_g = jnp.concatenate([path_indices, jnp.zeros((pad,), jnp.int32)])
    li_s = jnp.concatenate([link_indices, jnp.full((pad,), L, jnp.int32)])
    pi_s = jnp.concatenate([path_indices, jnp.full((pad,), P, jnp.int32)])
    pi_s2 = jnp.concatenate([
        jnp.where(pi_s < PH, pi_s, PH),
        jnp.where((pi_s >= PH) & (pi_s < 2 * PH), pi_s - PH, PH)])
    li_s2 = jnp.concatenate([
        jnp.where(li_s < LH, li_s, LH),
        jnp.where((li_s >= LH) & (li_s < 2 * LH), li_s - LH, LH)])
    li_g = li_g.reshape(EP // GCHUNK, GCHUNK)
    pi_g = pi_g.reshape(EP // GCHUNK, GCHUNK)
    zz = jnp.zeros((CHUNK, D), jnp.float32)

    link_states = jnp.zeros((L, D), jnp.float32).at[:, 0].set(link_capacity)
    path_states = jnp.zeros((P, D), jnp.float32).at[:, 0].set(
        jnp.reshape(TM, (-1,)))

    gather_a, gather_b, scatter_p, scatter_l = _sc_kernels()
    for it in range(N_ITERS):
        lsA, psA = gather_a(link_states, path_states, li_g, pi_g)
        lsB, psB = gather_b(link_states, path_states, li_g, pi_g)
        m1A, m2A = _mlp(lsA, psA, msg_W1, msg_b1, msg_W2, msg_b2,
                        msg_W3, msg_b3)
        m1B, m2B = _mlp(lsB, psB, msg_W1, msg_b1, msg_W2, msg_b2,
                        msg_W3, msg_b3)
        pa = scatter_p(m1A, m1B, pi_s2, zz)
        path_states = _gru(pa, path_states, path_Wih, path_Whh,
                           path_bih, path_bhh, P, 10000)
        if it < N_ITERS - 1:
            la = scatter_l(m2A, m2B, li_s2, zz)
            link_states = _gru(la, link_states, link_Wih, link_Whh,
                               link_bih, link_bhh, L, 10000)
    return _readout(path_states, ro_W1, ro_b1, ro_W2, ro_b2, ro_W3, ro_b3,
                    blk=10000)
